# baseline (device time: 95658 ns/iter reference)
import os

import jax
import jax.numpy as jnp
from jax import lax
from jax.experimental import pallas as pl
from jax.experimental.pallas import tpu as pltpu

N_DEV = 16
N_TOK = 1024
D_IN = 512
D_OUT = 1024
E_LOCAL = 4

P = 4
Z = 4
PCH = N_TOK // P
ZCH = PCH // Z
HALF = D_OUT // 2


def kernel(x, router_W, route_idx, expert_W):
    del router_W

    def body(x_ref, idx_ref, w_ref, out_ref,
             a_fwd_buf, a_rev_buf, b_buf,
             a_fwd_send, a_fwd_recv, a_rev_send, a_rev_recv,
             b_rs_send, b_rs_recv, b_ag_send, b_ag_recv,
             c_fwd_send, c_fwd_recv, c_rev_send, c_rev_recv):
        p = lax.axis_index("i")
        z = p // P
        q = lax.rem(p, P)
        right = z * P + lax.rem(q + 1, P)
        left = z * P + lax.rem(q + 3, P)
        o = z ^ (z >> 1)
        no = lax.rem(o + 1, Z)
        znext = (no ^ (no >> 1)) * P + q

        acc = jnp.zeros((N_TOK, D_OUT), jnp.float32)
        for k in range(E_LOCAL):
            e = p * E_LOCAL + k
            mask = idx_ref[:, :] == e
            xm = jnp.where(mask, x_ref[:, :], 0.0)
            acc = acc + jnp.dot(xm, w_ref[k],
                                preferred_element_type=jnp.float32)
        out_ref[:, :] = acc

        if os.environ.get("KERNEL_COMPUTE_ONLY"):
            return

        def prow(c):
            return pl.ds(lax.rem(c + 2 * P, P) * PCH, PCH)

        for s in range(P - 1):
            fwd = pltpu.make_async_remote_copy(
                src_ref=out_ref.at[prow(q - s), pl.ds(0, HALF)],
                dst_ref=a_fwd_buf.at[s],
                send_sem=a_fwd_send.at[s], recv_sem=a_fwd_recv.at[s],
                device_id=(right,), device_id_type=pl.DeviceIdType.MESH,
            )
            rev = pltpu.make_async_remote_copy(
                src_ref=out_ref.at[prow(q + s), pl.ds(HALF, HALF)],
                dst_ref=a_rev_buf.at[s],
                send_sem=a_rev_send.at[s], recv_sem=a_rev_recv.at[s],
                device_id=(left,), device_id_type=pl.DeviceIdType.MESH,
            )
            fwd.start()
            rev.start()
            fwd.wait()
            rev.wait()
            rf = prow(q - s - 1)
            rr = prow(q + s + 1)
            out_ref[rf, pl.ds(0, HALF)] = (
                out_ref[rf, pl.ds(0, HALF)] + a_fwd_buf[s])
            out_ref[rr, pl.ds(HALF, HALF)] = (
                out_ref[rr, pl.ds(HALF, HALF)] + a_rev_buf[s])

        own0 = lax.rem(q + 1, P) * PCH
        own1 = lax.rem(q + 3, P) * PCH

        def zrow0(f):
            return pl.ds(own0 + lax.rem(f + 2 * Z, Z) * ZCH, ZCH)

        def zrow1(f):
            return pl.ds(own1 + lax.rem(f + 2 * Z, Z) * ZCH, ZCH)

        for s in range(Z - 1):
            f_send = o - s
            r0 = pltpu.make_async_remote_copy(
                src_ref=out_ref.at[zrow0(f_send), pl.ds(0, HALF)],
                dst_ref=b_buf.at[s, 0],
                send_sem=b_rs_send.at[s, 0], recv_sem=b_rs_recv.at[s, 0],
                device_id=(znext,), device_id_type=pl.DeviceIdType.MESH,
            )
            r1 = pltpu.make_async_remote_copy(
                src_ref=out_ref.at[zrow1(f_send), pl.ds(HALF, HALF)],
                dst_ref=b_buf.at[s, 1],
                send_sem=b_rs_send.at[s, 1], recv_sem=b_rs_recv.at[s, 1],
                device_id=(znext,), device_id_type=pl.DeviceIdType.MESH,
            )
            r0.start()
            r1.start()
            r0.wait()
            r1.wait()
            fr = o - s - 1
            out_ref[zrow0(fr), pl.ds(0, HALF)] = (
                out_ref[zrow0(fr), pl.ds(0, HALF)] + b_buf[s, 0])
            out_ref[zrow1(fr), pl.ds(HALF, HALF)] = (
                out_ref[zrow1(fr), pl.ds(HALF, HALF)] + b_buf[s, 1])

        for s in range(Z - 1):
            g = o + 1 - s
            r0 = pltpu.make_async_remote_copy(
                src_ref=out_ref.at[zrow0(g), pl.ds(0, HALF)],
                dst_ref=out_ref.at[zrow0(g), pl.ds(0, HALF)],
                send_sem=b_ag_send.at[s, 0], recv_sem=b_ag_recv.at[s, 0],
                device_id=(znext,), device_id_type=pl.DeviceIdType.MESH,
            )
            r1 = pltpu.make_async_remote_copy(
                src_ref=out_ref.at[zrow1(g), pl.ds(HALF, HALF)],
                dst_ref=out_ref.at[zrow1(g), pl.ds(HALF, HALF)],
                send_sem=b_ag_send.at[s, 1], recv_sem=b_ag_recv.at[s, 1],
                device_id=(znext,), device_id_type=pl.DeviceIdType.MESH,
            )
            r0.start()
            r1.start()
            r0.wait()
            r1.wait()

        for s in range(P - 1):
            fwd = pltpu.make_async_remote_copy(
                src_ref=out_ref.at[prow(q + 1 - s), pl.ds(0, HALF)],
                dst_ref=out_ref.at[prow(q + 1 - s), pl.ds(0, HALF)],
                send_sem=c_fwd_send.at[s], recv_sem=c_fwd_recv.at[s],
                device_id=(right,), device_id_type=pl.DeviceIdType.MESH,
            )
            rev = pltpu.make_async_remote_copy(
                src_ref=out_ref.at[prow(q - 1 + s), pl.ds(HALF, HALF)],
                dst_ref=out_ref.at[prow(q - 1 + s), pl.ds(HALF, HALF)],
                send_sem=c_rev_send.at[s], recv_sem=c_rev_recv.at[s],
                device_id=(left,), device_id_type=pl.DeviceIdType.MESH,
            )
            fwd.start()
            rev.start()
            fwd.wait()
            rev.wait()

    return pl.pallas_call(
        body,
        out_shape=jax.ShapeDtypeStruct((N_TOK, D_OUT), jnp.float32),
        in_specs=[pl.BlockSpec(memory_space=pltpu.VMEM)] * 3,
        out_specs=pl.BlockSpec(memory_space=pltpu.VMEM),
        scratch_shapes=[
            pltpu.VMEM((P - 1, PCH, HALF), jnp.float32),
            pltpu.VMEM((P - 1, PCH, HALF), jnp.float32),
            pltpu.VMEM((Z - 1, 2, ZCH, HALF), jnp.float32),
            pltpu.SemaphoreType.DMA((P - 1,)),
            pltpu.SemaphoreType.DMA((P - 1,)),
            pltpu.SemaphoreType.DMA((P - 1,)),
            pltpu.SemaphoreType.DMA((P - 1,)),
            pltpu.SemaphoreType.DMA((Z - 1, 2)),
            pltpu.SemaphoreType.DMA((Z - 1, 2)),
            pltpu.SemaphoreType.DMA((Z - 1, 2)),
            pltpu.SemaphoreType.DMA((Z - 1, 2)),
            pltpu.SemaphoreType.DMA((P - 1,)),
            pltpu.SemaphoreType.DMA((P - 1,)),
            pltpu.SemaphoreType.DMA((P - 1,)),
            pltpu.SemaphoreType.DMA((P - 1,)),
        ],
    )(x, route_idx, expert_W)


# device time: 92558 ns/iter; 1.0335x vs baseline; 1.0335x over previous
import os

import jax
import jax.numpy as jnp
from jax import lax
from jax.experimental import pallas as pl
from jax.experimental.pallas import tpu as pltpu

N_DEV = 16
N_TOK = 1024
D_IN = 512
D_OUT = 1024
E_LOCAL = 4

P = 4
Z = 4
PCH = N_TOK // P
ZCH = PCH // Z
HALF = D_OUT // 2


def kernel(x, router_W, route_idx, expert_W):
    del router_W

    def body(x_ref, idx_ref, w_ref, out_ref,
             a_fwd_buf, a_rev_buf, b_buf,
             a_fwd_send, a_fwd_recv, a_rev_send, a_rev_recv,
             b_rs_send, b_rs_recv, b_ag_send, b_ag_recv,
             c_fwd_send, c_fwd_recv, c_rev_send, c_rev_recv):
        p = lax.axis_index("i")
        z = p // P
        q = lax.rem(p, P)
        right = z * P + lax.rem(q + 1, P)
        left = z * P + lax.rem(q + 3, P)
        o = z ^ (z >> 1)
        no = lax.rem(o + 1, Z)
        znext = (no ^ (no >> 1)) * P + q

        sends = []

        def prow(c):
            return pl.ds(lax.rem(c + 2 * P, P) * PCH, PCH)

        def compute_chunk(c):
            rows = prow(c)
            xc = x_ref[rows, :]
            ic = idx_ref[rows, :]
            acc = jnp.zeros((PCH, D_OUT), jnp.float32)
            for k in range(E_LOCAL):
                e = p * E_LOCAL + k
                xm = jnp.where(ic == e, xc, 0.0)
                acc = acc + jnp.dot(xm, w_ref[k],
                                    preferred_element_type=jnp.float32)
            out_ref[rows, :] = acc

        compute_chunk(q)

        if os.environ.get("KERNEL_COMPUTE_ONLY"):
            for c in range(1, P):
                compute_chunk(q + c)
            return

        for s in range(P - 1):
            fwd = pltpu.make_async_remote_copy(
                src_ref=out_ref.at[prow(q - s), pl.ds(0, HALF)],
                dst_ref=a_fwd_buf.at[s],
                send_sem=a_fwd_send.at[s], recv_sem=a_fwd_recv.at[s],
                device_id=(right,), device_id_type=pl.DeviceIdType.MESH,
            )
            rev = pltpu.make_async_remote_copy(
                src_ref=out_ref.at[prow(q + s), pl.ds(HALF, HALF)],
                dst_ref=a_rev_buf.at[s],
                send_sem=a_rev_send.at[s], recv_sem=a_rev_recv.at[s],
                device_id=(left,), device_id_type=pl.DeviceIdType.MESH,
            )
            fwd.start()
            rev.start()
            sends += [fwd, rev]
            if s == 0:
                compute_chunk(q + 3)
                compute_chunk(q + 1)
            elif s == 1:
                compute_chunk(q + 2)
            rf = prow(q - s - 1)
            rr = prow(q + s + 1)
            fwd.wait_recv()
            out_ref[rf, pl.ds(0, HALF)] = (
                out_ref[rf, pl.ds(0, HALF)] + a_fwd_buf[s])
            rev.wait_recv()
            out_ref[rr, pl.ds(HALF, HALF)] = (
                out_ref[rr, pl.ds(HALF, HALF)] + a_rev_buf[s])

        own0 = lax.rem(q + 1, P) * PCH
        own1 = lax.rem(q + 3, P) * PCH

        def zrow0(f):
            return pl.ds(own0 + lax.rem(f + 2 * Z, Z) * ZCH, ZCH)

        def zrow1(f):
            return pl.ds(own1 + lax.rem(f + 2 * Z, Z) * ZCH, ZCH)

        for s in range(Z - 1):
            f_send = o - s
            r0 = pltpu.make_async_remote_copy(
                src_ref=out_ref.at[zrow0(f_send), pl.ds(0, HALF)],
                dst_ref=b_buf.at[s, 0],
                send_sem=b_rs_send.at[s, 0], recv_sem=b_rs_recv.at[s, 0],
                device_id=(znext,), device_id_type=pl.DeviceIdType.MESH,
            )
            r1 = pltpu.make_async_remote_copy(
                src_ref=out_ref.at[zrow1(f_send), pl.ds(HALF, HALF)],
                dst_ref=b_buf.at[s, 1],
                send_sem=b_rs_send.at[s, 1], recv_sem=b_rs_recv.at[s, 1],
                device_id=(znext,), device_id_type=pl.DeviceIdType.MESH,
            )
            r0.start()
            r1.start()
            sends += [r0, r1]
            fr = o - s - 1
            r0.wait_recv()
            out_ref[zrow0(fr), pl.ds(0, HALF)] = (
                out_ref[zrow0(fr), pl.ds(0, HALF)] + b_buf[s, 0])
            r1.wait_recv()
            out_ref[zrow1(fr), pl.ds(HALF, HALF)] = (
                out_ref[zrow1(fr), pl.ds(HALF, HALF)] + b_buf[s, 1])

        for s in range(Z - 1):
            g = o + 1 - s
            r0 = pltpu.make_async_remote_copy(
                src_ref=out_ref.at[zrow0(g), pl.ds(0, HALF)],
                dst_ref=out_ref.at[zrow0(g), pl.ds(0, HALF)],
                send_sem=b_ag_send.at[s, 0], recv_sem=b_ag_recv.at[s, 0],
                device_id=(znext,), device_id_type=pl.DeviceIdType.MESH,
            )
            r1 = pltpu.make_async_remote_copy(
                src_ref=out_ref.at[zrow1(g), pl.ds(HALF, HALF)],
                dst_ref=out_ref.at[zrow1(g), pl.ds(HALF, HALF)],
                send_sem=b_ag_send.at[s, 1], recv_sem=b_ag_recv.at[s, 1],
                device_id=(znext,), device_id_type=pl.DeviceIdType.MESH,
            )
            r0.start()
            r1.start()
            sends += [r0, r1]
            r0.wait_recv()
            r1.wait_recv()

        for s in range(P - 1):
            fwd = pltpu.make_async_remote_copy(
                src_ref=out_ref.at[prow(q + 1 - s), pl.ds(0, HALF)],
                dst_ref=out_ref.at[prow(q + 1 - s), pl.ds(0, HALF)],
                send_sem=c_fwd_send.at[s], recv_sem=c_fwd_recv.at[s],
                device_id=(right,), device_id_type=pl.DeviceIdType.MESH,
            )
            rev = pltpu.make_async_remote_copy(
                src_ref=out_ref.at[prow(q - 1 + s), pl.ds(HALF, HALF)],
                dst_ref=out_ref.at[prow(q - 1 + s), pl.ds(HALF, HALF)],
                send_sem=c_rev_send.at[s], recv_sem=c_rev_recv.at[s],
                device_id=(left,), device_id_type=pl.DeviceIdType.MESH,
            )
            fwd.start()
            rev.start()
            sends += [fwd, rev]
            fwd.wait_recv()
            rev.wait_recv()

        for d in sends:
            d.wait_send()

    return pl.pallas_call(
        body,
        out_shape=jax.ShapeDtypeStruct((N_TOK, D_OUT), jnp.float32),
        in_specs=[pl.BlockSpec(memory_space=pltpu.VMEM)] * 3,
        out_specs=pl.BlockSpec(memory_space=pltpu.VMEM),
        scratch_shapes=[
            pltpu.VMEM((P - 1, PCH, HALF), jnp.float32),
            pltpu.VMEM((P - 1, PCH, HALF), jnp.float32),
            pltpu.VMEM((Z - 1, 2, ZCH, HALF), jnp.float32),
            pltpu.SemaphoreType.DMA((P - 1,)),
            pltpu.SemaphoreType.DMA((P - 1,)),
            pltpu.SemaphoreType.DMA((P - 1,)),
            pltpu.SemaphoreType.DMA((P - 1,)),
            pltpu.SemaphoreType.DMA((Z - 1, 2)),
            pltpu.SemaphoreType.DMA((Z - 1, 2)),
            pltpu.SemaphoreType.DMA((Z - 1, 2)),
            pltpu.SemaphoreType.DMA((Z - 1, 2)),
            pltpu.SemaphoreType.DMA((P - 1,)),
            pltpu.SemaphoreType.DMA((P - 1,)),
            pltpu.SemaphoreType.DMA((P - 1,)),
            pltpu.SemaphoreType.DMA((P - 1,)),
        ],
    )(x, route_idx, expert_W)
